# Initial kernel scaffold; baseline (speedup 1.0000x reference)
import jax
import jax.numpy as jnp
from jax.experimental import pallas as pl


def kernel(x_nchw, w1, b1, w2, b2):
    raise NotImplementedError("write your pallas kernel here")


# speedup vs baseline: 8.0986x; 8.0986x over previous
import functools

import jax
import jax.numpy as jnp
from jax.experimental import pallas as pl
from jax.experimental.pallas import tpu as pltpu


def _glu_body(e0_ref, od_ref, e1_ref, w_ref, b_ref, o_ref, *, wo, stride, cin,
              cout):
    w0 = w_ref[0]
    w1 = w_ref[1]
    w2 = w_ref[2]
    bias = b_ref[...]
    kwin = w_ref.shape[2]
    step = stride * cin

    for ow in range(wo):
        s = ow * step
        acc = jnp.dot(w0, e0_ref[0, pl.ds(s, kwin), :],
                      preferred_element_type=jnp.float32)
        acc = acc + jnp.dot(w1, od_ref[0, pl.ds(s, kwin), :],
                            preferred_element_type=jnp.float32)
        acc = acc + jnp.dot(w2, e1_ref[0, pl.ds(s, kwin), :],
                            preferred_element_type=jnp.float32)
        acc = acc + bias
        lin = acc[:cout, :]
        g = acc[cout:, :]
        z = jnp.exp(-jnp.abs(g))
        gate = jnp.where(g >= 0, 1.0, z) / (1.0 + z)
        o_ref[0, ow] = (lin * gate).astype(o_ref.dtype)


@functools.partial(jax.jit, static_argnames=("stride",))
def _glu_conv2d(x_nchw, w1, b1, w2, b2, *, stride):
    cout, cin, kh, kw = w1.shape
    n, _, h, w = x_nchw.shape
    ho = (h - kh) // stride + 1
    wo = (w - kw) // stride + 1

    x_r = jnp.transpose(x_nchw, (2, 3, 1, 0)).reshape(h, w * cin, n)
    x_even = x_r[0::2]
    x_odd = x_r[1::2]

    w_cat = jnp.concatenate([w1, w2], axis=0)
    w_g = jnp.transpose(w_cat, (2, 0, 3, 1)).reshape(kh, 2 * cout, kw * cin)
    b_cat = jnp.concatenate([b1, b2]).reshape(2 * cout, 1)

    body = functools.partial(_glu_body, wo=wo, stride=stride, cin=cin,
                             cout=cout)

    flops = 2 * n * ho * wo * 2 * cout * kh * kw * cin + 8 * n * cout * ho * wo
    cost = pl.CostEstimate(
        flops=flops,
        transcendentals=n * cout * ho * wo,
        bytes_accessed=4 * (n * cin * h * w + kh * 2 * cout * kw * cin
                            + 2 * cout + n * cout * ho * wo),
    )

    out = pl.pallas_call(
        body,
        out_shape=jax.ShapeDtypeStruct((ho, wo, cout, n), jnp.float32),
        grid=(ho,),
        in_specs=[
            pl.BlockSpec((1, w * cin, n), lambda i: (i, 0, 0)),
            pl.BlockSpec((1, w * cin, n), lambda i: (i, 0, 0)),
            pl.BlockSpec((1, w * cin, n), lambda i: (i + 1, 0, 0)),
            pl.BlockSpec((kh, 2 * cout, kw * cin), lambda i: (0, 0, 0),
                         pipeline_mode=pl.Buffered(1)),
            pl.BlockSpec((2 * cout, 1), lambda i: (0, 0),
                         pipeline_mode=pl.Buffered(1)),
        ],
        out_specs=pl.BlockSpec((1, wo, cout, n), lambda i: (i, 0, 0, 0)),
        compiler_params=pltpu.CompilerParams(
            dimension_semantics=("parallel",),
            vmem_limit_bytes=64 * 1024 * 1024,
        ),
        cost_estimate=cost,
    )(x_even, x_odd, x_even, w_g, b_cat)

    return jnp.transpose(out, (3, 2, 0, 1))


def kernel(x_nchw, w1, b1, w2, b2):
    return _glu_conv2d(x_nchw, w1, b1, w2, b2, stride=2)


# speedup vs baseline: 10.3068x; 1.2727x over previous
import functools

import jax
import jax.numpy as jnp
from jax.experimental import pallas as pl
from jax.experimental.pallas import tpu as pltpu


def _glu_body(e0_ref, od_ref, e1_ref, w_ref, b_ref, o_ref, *, wo, stride, cin,
              cout):
    w0 = w_ref[0]
    w1 = w_ref[1]
    w2 = w_ref[2]
    bias = b_ref[...]
    kwin = w_ref.shape[2]
    step = stride * cin

    for ow in range(wo):
        s = ow * step
        acc = jnp.dot(w0, e0_ref[0, pl.ds(s, kwin), :],
                      preferred_element_type=jnp.float32)
        acc = acc + jnp.dot(w1, od_ref[0, pl.ds(s, kwin), :],
                            preferred_element_type=jnp.float32)
        acc = acc + jnp.dot(w2, e1_ref[0, pl.ds(s, kwin), :],
                            preferred_element_type=jnp.float32)
        acc = acc + bias
        lin = acc[:cout, :]
        g = acc[cout:, :]
        z = jnp.exp(-jnp.abs(g))
        gate = jnp.where(g >= 0, 1.0, z) / (1.0 + z)
        o_ref[0, ow] = (lin * gate).astype(o_ref.dtype)


@functools.partial(jax.jit, static_argnames=("stride",))
def _glu_conv2d(x_nchw, w1, b1, w2, b2, *, stride):
    cout, cin, kh, kw = w1.shape
    n, _, h, w = x_nchw.shape
    ho = (h - kh) // stride + 1
    wo = (w - kw) // stride + 1

    x_r = jnp.transpose(x_nchw.astype(jnp.bfloat16),
                        (2, 3, 1, 0)).reshape(h, w * cin, n)

    w_cat = jnp.concatenate([w1, w2], axis=0).astype(jnp.bfloat16)
    w_g = jnp.transpose(w_cat, (2, 0, 3, 1)).reshape(kh, 2 * cout, kw * cin)
    b_cat = jnp.concatenate([b1, b2]).reshape(2 * cout, 1)

    body = functools.partial(_glu_body, wo=wo, stride=stride, cin=cin,
                             cout=cout)

    flops = 2 * n * ho * wo * 2 * cout * kh * kw * cin + 8 * n * cout * ho * wo
    cost = pl.CostEstimate(
        flops=flops,
        transcendentals=n * cout * ho * wo,
        bytes_accessed=4 * (n * cin * h * w + kh * 2 * cout * kw * cin
                            + 2 * cout + n * cout * ho * wo),
    )

    out = pl.pallas_call(
        body,
        out_shape=jax.ShapeDtypeStruct((ho, wo, cout, n), jnp.float32),
        grid=(ho,),
        in_specs=[
            pl.BlockSpec((1, w * cin, n), lambda i: (stride * i, 0, 0)),
            pl.BlockSpec((1, w * cin, n), lambda i: (stride * i + 1, 0, 0)),
            pl.BlockSpec((1, w * cin, n), lambda i: (stride * i + 2, 0, 0)),
            pl.BlockSpec((kh, 2 * cout, kw * cin), lambda i: (0, 0, 0),
                         pipeline_mode=pl.Buffered(1)),
            pl.BlockSpec((2 * cout, 1), lambda i: (0, 0),
                         pipeline_mode=pl.Buffered(1)),
        ],
        out_specs=pl.BlockSpec((1, wo, cout, n), lambda i: (i, 0, 0, 0)),
        compiler_params=pltpu.CompilerParams(
            dimension_semantics=("parallel",),
            vmem_limit_bytes=64 * 1024 * 1024,
        ),
        cost_estimate=cost,
    )(x_r, x_r, x_r, w_g, b_cat)

    return jnp.transpose(out, (3, 2, 0, 1))


def kernel(x_nchw, w1, b1, w2, b2):
    return _glu_conv2d(x_nchw, w1, b1, w2, b2, stride=2)
